# baseline (speedup 1.0000x reference)
import functools

import jax
import jax.numpy as jnp
from jax import lax
from jax.experimental import pallas as pl
from jax.experimental.pallas import tpu as pltpu
from jax.experimental.pallas import tpu_sc as plsc

N = 10000
E = 320000
IN_CH = 128
HID = 128
OUT_CH = 64

NC = 2
NS = 16
NW = NC * NS
EPW = E // NW
CH = 80
NCHUNK = EPW // CH
ROWB = 80
NROWCH = N // ROWB

_MESH = plsc.VectorSubcoreMesh(
    core_axis_name="c", subcore_axis_name="s", num_cores=NC, num_subcores=NS)


def _fill(buf, rows, width, value):
  vec = jnp.full((16,), value, jnp.float32)

  def body(r, carry):
    for j in range(width // 16):
      buf[r, pl.ds(j * 16, 16)] = vec
    return carry

  lax.fori_loop(0, rows, body, 0)


_DEGW = 8


def _sc_deg(eidx):
  @functools.partial(
      pl.kernel,
      out_type=jax.ShapeDtypeStruct((NC, N, 16), jnp.float32),
      mesh=_MESH,
      compiler_params=pltpu.CompilerParams(use_tc_tiling_on_sc=False),
      scratch_types=[
          pltpu.VMEM((EPW,), jnp.int32),
          pltpu.VMEM((ROWB, 16), jnp.float32),
          pltpu.VMEM((ROWB, 16), jnp.float32),
          pltpu.VMEM_SHARED((N, 16), jnp.float32),
          pltpu.SemaphoreType.DMA,
          pltpu.SemaphoreType.DMA,
      ],
  )
  def k(e_hbm, out_hbm, idxd, zrows, ones, acc, psem, ssem):
    c = lax.axis_index("c")
    s = lax.axis_index("s")
    gid = c * NS + s

    cp_idx = pltpu.async_copy(e_hbm.at[1, gid], idxd, psem)
    _fill(zrows, ROWB, 16, 0.0)
    _fill(ones, ROWB, 16, 1.0)

    def zacc(j, carry):
      @pl.when(lax.rem(j, NS) == s)
      def _():
        pltpu.sync_copy(zrows, acc.at[pl.ds(j * ROWB, ROWB)])
      return carry

    lax.fori_loop(0, NROWCH, zacc, 0)
    cp_idx.wait()
    plsc.subcore_barrier()

    def step(j, carry):
      jj = pl.ds(j * CH, CH)
      pltpu.async_copy(ones, acc.at[idxd.at[jj]], ssem, add=True)

      @pl.when(j >= _DEGW)
      def _():
        pltpu.make_async_copy(ones, acc.at[idxd.at[jj]], ssem).wait()
      return carry

    lax.fori_loop(0, NCHUNK, step, 0)

    def drain(j, carry):
      pltpu.make_async_copy(ones, acc.at[idxd.at[pl.ds(0, CH)]], ssem).wait()
      return carry

    lax.fori_loop(0, _DEGW, drain, 0)
    plsc.subcore_barrier()

    def cpout(j, carry):
      @pl.when(lax.rem(j, NS) == s)
      def _():
        pltpu.sync_copy(acc.at[pl.ds(j * ROWB, ROWB)], zrows)
        pltpu.sync_copy(zrows, out_hbm.at[c, pl.ds(j * ROWB, ROWB)])
      return carry

    lax.fori_loop(0, NROWCH, cpout, 0)

  return k(eidx)


def _sc_spmm(table, eidx, width, nb, ch):
  nch = EPW // ch

  @functools.partial(
      pl.kernel,
      out_type=jax.ShapeDtypeStruct((NC, N, width), jnp.float32),
      mesh=_MESH,
      compiler_params=pltpu.CompilerParams(use_tc_tiling_on_sc=False),
      scratch_types=[
          pltpu.VMEM((EPW,), jnp.int32),
          pltpu.VMEM((EPW,), jnp.int32),
      ] + [pltpu.VMEM((ch, width), jnp.float32)] * nb + [
          pltpu.VMEM_SHARED((N, width), jnp.float32),
          pltpu.SemaphoreType.DMA,
      ] + [pltpu.SemaphoreType.DMA] * (2 * nb),
  )
  def k(table_hbm, e_hbm, out_hbm, idxs, idxd, *bufs):
    rows = bufs[:nb]
    acc = bufs[nb]
    psem = bufs[nb + 1]
    g = bufs[nb + 2:nb + 2 + nb]
    st = bufs[nb + 2 + nb:]
    c = lax.axis_index("c")
    s = lax.axis_index("s")
    gid = c * NS + s

    cp_si = pltpu.async_copy(e_hbm.at[0, gid], idxs, psem)
    cp_di = pltpu.async_copy(e_hbm.at[1, gid], idxd, psem)

    _fill(rows[0], ch, width, 0.0)

    def zacc(j, carry):
      @pl.when(lax.rem(j, NS) == s)
      def _():
        pltpu.sync_copy(rows[0], acc.at[pl.ds(j * ch, ch)])
      return carry

    lax.fori_loop(0, N // ch, zacc, 0)
    cp_si.wait()
    cp_di.wait()
    for b in range(nb - 1):
      pltpu.async_copy(table_hbm.at[idxs.at[pl.ds(b * ch, ch)]], rows[b], g[b])
    plsc.subcore_barrier()

    def step(i, carry):
      for b in range(nb):
        j = nb * i + b
        js = pl.ds(j * ch, ch)
        ja = pl.ds((j + nb - 1) * ch, ch)
        pltpu.make_async_copy(table_hbm.at[idxs.at[js]], rows[b], g[b]).wait()
        bn = (b + nb - 1) % nb
        if b == 0:
          @pl.when(i > 0)
          def _():
            pltpu.make_async_copy(rows[bn], acc.at[idxd.at[js]], st[bn]).wait()
          pltpu.async_copy(table_hbm.at[idxs.at[ja]], rows[bn], g[bn])
        else:
          pltpu.make_async_copy(rows[bn], acc.at[idxd.at[js]], st[bn]).wait()

          @pl.when(i < nch // nb - 1)
          def _():
            pltpu.async_copy(table_hbm.at[idxs.at[ja]], rows[bn], g[bn])
        pltpu.async_copy(rows[b], acc.at[idxd.at[js]], st[b], add=True)
      return carry

    lax.fori_loop(0, nch // nb, step, 0)
    pltpu.make_async_copy(rows[nb - 1], acc.at[idxd.at[pl.ds(0, ch)]],
                          st[nb - 1]).wait()
    plsc.subcore_barrier()

    def cpout(j, carry):
      @pl.when(lax.rem(j, NS) == s)
      def _():
        pltpu.sync_copy(acc.at[pl.ds(j * ch, ch)], rows[0])
        pltpu.sync_copy(rows[0], out_hbm.at[c, pl.ds(j * ch, ch)])
      return carry

    lax.fori_loop(0, N // ch, cpout, 0)

  return k(table, eidx)


_NQ = OUT_CH // 16


_NBD = 5


def _sc_edge_decode(u, z, eidx, bb16):
  @functools.partial(
      pl.kernel,
      out_type=jax.ShapeDtypeStruct((E,), jnp.float32),
      mesh=_MESH,
      compiler_params=pltpu.CompilerParams(
          use_tc_tiling_on_sc=False, needs_layout_passes=False),
      scratch_types=[
          pltpu.VMEM((EPW,), jnp.int32),
          pltpu.VMEM((EPW,), jnp.int32),
          pltpu.VMEM((16,), jnp.float32),
      ] + [pltpu.VMEM((CH, OUT_CH), jnp.float32)] * (2 * _NBD)
        + [pltpu.VMEM((CH,), jnp.float32)] * _NBD
        + [pltpu.SemaphoreType.DMA] * (1 + 3 * _NBD),
  )
  def k(u_hbm, z_hbm, e_hbm, bb_hbm, out_hbm, idxs, idxd, bbv, *bufs):
    ub = bufs[0:_NBD]
    zb = bufs[_NBD:2 * _NBD]
    ob = bufs[2 * _NBD:3 * _NBD]
    psem = bufs[3 * _NBD]
    gu = bufs[3 * _NBD + 1:4 * _NBD + 1]
    gz = bufs[4 * _NBD + 1:5 * _NBD + 1]
    wo = bufs[5 * _NBD + 1:6 * _NBD + 1]
    c = lax.axis_index("c")
    s = lax.axis_index("s")
    gid = c * NS + s
    base = gid * EPW

    cp_si = pltpu.async_copy(e_hbm.at[0, gid], idxs, psem)
    cp_di = pltpu.async_copy(e_hbm.at[1, gid], idxd, psem)
    pltpu.sync_copy(bb_hbm, bbv)
    bias = bbv[...]
    lane = lax.iota(jnp.int32, 16)
    cp_si.wait()
    cp_di.wait()
    for b in range(_NBD - 1):
      bs = pl.ds(b * CH, CH)
      pltpu.async_copy(u_hbm.at[idxs.at[bs]], ub[b], gu[b])
      pltpu.async_copy(z_hbm.at[idxd.at[bs]], zb[b], gz[b])

    def dot_chunk(ubuf, zbuf, obuf):
      def grp(g, carry):
        res = jnp.zeros((16,), jnp.float32)
        for e in range(16):
          row = g * 16 + e
          acc = ubuf[row, pl.ds(0, 16)] * zbuf[row, pl.ds(0, 16)]
          for q in range(1, _NQ):
            acc = acc + ubuf[row, pl.ds(q * 16, 16)] * zbuf[row, pl.ds(q * 16, 16)]
          res = jnp.where(lane == e, jnp.full((16,), jnp.sum(acc)), res)
        obuf[pl.ds(g * 16, 16)] = 1.0 / (1.0 + jnp.exp(-(res + bias)))
        return carry

      lax.fori_loop(0, CH // 16, grp, 0)

    def step(i, carry):
      for b in range(_NBD):
        j = _NBD * i + b
        js = pl.ds(j * CH, CH)
        ja = pl.ds((j + _NBD - 1) * CH, CH)
        off = pl.multiple_of(base + j * CH, 8)
        pltpu.make_async_copy(u_hbm.at[idxs.at[js]], ub[b], gu[b]).wait()
        pltpu.make_async_copy(z_hbm.at[idxd.at[js]], zb[b], gz[b]).wait()
        bn = (b + _NBD - 1) % _NBD
        if b == 0:
          pltpu.async_copy(u_hbm.at[idxs.at[ja]], ub[bn], gu[bn])
          pltpu.async_copy(z_hbm.at[idxd.at[ja]], zb[bn], gz[bn])
        else:
          @pl.when(i < NCHUNK // _NBD - 1)
          def _():
            pltpu.async_copy(u_hbm.at[idxs.at[ja]], ub[bn], gu[bn])
            pltpu.async_copy(z_hbm.at[idxd.at[ja]], zb[bn], gz[bn])

        @pl.when(i > 0)
        def _():
          pltpu.make_async_copy(ob[b], out_hbm.at[pl.ds(off, CH)], wo[b]).wait()

        dot_chunk(ub[b], zb[b], ob[b])
        pltpu.async_copy(ob[b], out_hbm.at[pl.ds(off, CH)], wo[b])
      return carry

    lax.fori_loop(0, NCHUNK // _NBD, step, 0)
    for b in range(_NBD):
      pltpu.make_async_copy(ob[b], out_hbm.at[pl.ds(base, CH)], wo[b]).wait()

  return k(u, z, eidx, bb16)


_NBLK = 2000
_GRID = N // _NBLK


def _dinv_from(deg_ref):
  deg = 1.0 + deg_ref[0, :, 0:1] + deg_ref[1, :, 0:1]
  return lax.rsqrt(deg)


_DEGSPEC = pl.BlockSpec((NC, _NBLK, 16), lambda i: (0, i, 0))


def _tc_a(x, W1, deg2):
  def body(x_ref, w_ref, d_ref, q_ref):
    dinv = _dinv_from(d_ref)
    p = jnp.dot(x_ref[:, :], w_ref[:, :], preferred_element_type=jnp.float32)
    q_ref[:, :] = dinv * p

  return pl.pallas_call(
      body,
      grid=(_GRID,),
      in_specs=[
          pl.BlockSpec((_NBLK, IN_CH), lambda i: (i, 0)),
          pl.BlockSpec((IN_CH, HID), lambda i: (0, 0)),
          _DEGSPEC,
      ],
      out_specs=pl.BlockSpec((_NBLK, HID), lambda i: (i, 0)),
      out_shape=jax.ShapeDtypeStruct((N, HID), jnp.float32),
  )(x, W1, deg2)


def _tc_c(s1, q1, deg2, b1, W2):
  def body(s_ref, q_ref, d_ref, bias_ref, w_ref, out_ref):
    dinv = _dinv_from(d_ref)
    h = dinv * (s_ref[0] + s_ref[1] + q_ref[:, :]) + bias_ref[:, :]
    h = jnp.maximum(h, 0.0)
    p2 = jnp.dot(h, w_ref[:, :], preferred_element_type=jnp.float32)
    out_ref[:, :] = dinv * p2

  return pl.pallas_call(
      body,
      grid=(_GRID,),
      in_specs=[
          pl.BlockSpec((NC, _NBLK, HID), lambda i: (0, i, 0)),
          pl.BlockSpec((_NBLK, HID), lambda i: (i, 0)),
          _DEGSPEC,
          pl.BlockSpec((1, HID), lambda i: (0, 0)),
          pl.BlockSpec((HID, OUT_CH), lambda i: (0, 0)),
      ],
      out_specs=pl.BlockSpec((_NBLK, OUT_CH), lambda i: (i, 0)),
      out_shape=jax.ShapeDtypeStruct((N, OUT_CH), jnp.float32),
  )(s1, q1, deg2, b1, W2)


def _tc_e(s2, q2, deg2, b2, Wb0):
  def body(s_ref, q_ref, d_ref, bias_ref, w_ref, z_ref, u_ref):
    dinv = _dinv_from(d_ref)
    z = dinv * (s_ref[0] + s_ref[1] + q_ref[:, :]) + bias_ref[:, :]
    z_ref[:, :] = z
    u_ref[:, :] = jnp.dot(z, w_ref[:, :], preferred_element_type=jnp.float32)

  return pl.pallas_call(
      body,
      grid=(_GRID,),
      in_specs=[
          pl.BlockSpec((NC, _NBLK, OUT_CH), lambda i: (0, i, 0)),
          pl.BlockSpec((_NBLK, OUT_CH), lambda i: (i, 0)),
          _DEGSPEC,
          pl.BlockSpec((1, OUT_CH), lambda i: (0, 0)),
          pl.BlockSpec((OUT_CH, OUT_CH), lambda i: (0, 0)),
      ],
      out_specs=[
          pl.BlockSpec((_NBLK, OUT_CH), lambda i: (i, 0)),
          pl.BlockSpec((_NBLK, OUT_CH), lambda i: (i, 0)),
      ],
      out_shape=[
          jax.ShapeDtypeStruct((N, OUT_CH), jnp.float32),
          jax.ShapeDtypeStruct((N, OUT_CH), jnp.float32),
      ],
  )(s2, q2, deg2, b2, Wb0)


def kernel(x, edge_index, W1, b1, W2, b2, Wb, bb):
  eidx = edge_index.reshape(2, NW, EPW)

  deg2 = _sc_deg(eidx)
  q1 = _tc_a(x, W1, deg2)
  s1 = _sc_spmm(q1, eidx, HID, 5, 40)
  q2 = _tc_c(s1, q1, deg2, b1.reshape(1, HID), W2)
  s2 = _sc_spmm(q2, eidx, OUT_CH, 5, 80)
  z, u = _tc_e(s2, q2, deg2, b2.reshape(1, OUT_CH), Wb[0])
  bb16 = jnp.broadcast_to(bb.reshape(1), (16,))
  return _sc_edge_decode(u, z, eidx, bb16).reshape(E, 1)


# speedup vs baseline: 1.0438x; 1.0438x over previous
import functools

import jax
import jax.numpy as jnp
from jax import lax
from jax.experimental import pallas as pl
from jax.experimental.pallas import tpu as pltpu
from jax.experimental.pallas import tpu_sc as plsc

N = 10000
E = 320000
IN_CH = 128
HID = 128
OUT_CH = 64

NC = 2
NS = 16
NW = NC * NS
EPW = E // NW
CH = 80
NCHUNK = EPW // CH
ROWB = 80
NROWCH = N // ROWB

_MESH = plsc.VectorSubcoreMesh(
    core_axis_name="c", subcore_axis_name="s", num_cores=NC, num_subcores=NS)


def _fill(buf, rows, width, value):
  vec = jnp.full((16,), value, jnp.float32)

  def body(r, carry):
    for j in range(width // 16):
      buf[r, pl.ds(j * 16, 16)] = vec
    return carry

  lax.fori_loop(0, rows, body, 0)


_DEGW = 8


def _sc_deg(eidx):
  @functools.partial(
      pl.kernel,
      out_type=jax.ShapeDtypeStruct((NC, N, 16), jnp.float32),
      mesh=_MESH,
      compiler_params=pltpu.CompilerParams(use_tc_tiling_on_sc=False),
      scratch_types=[
          pltpu.VMEM((EPW,), jnp.int32),
          pltpu.VMEM((ROWB, 16), jnp.float32),
          pltpu.VMEM((ROWB, 16), jnp.float32),
          pltpu.VMEM_SHARED((N, 16), jnp.float32),
          pltpu.SemaphoreType.DMA,
          pltpu.SemaphoreType.DMA,
      ],
  )
  def k(e_hbm, out_hbm, idxd, zrows, ones, acc, psem, ssem):
    c = lax.axis_index("c")
    s = lax.axis_index("s")
    gid = c * NS + s

    cp_idx = pltpu.async_copy(e_hbm.at[1, gid], idxd, psem)
    _fill(zrows, ROWB, 16, 0.0)
    _fill(ones, ROWB, 16, 1.0)

    def zacc(j, carry):
      @pl.when(lax.rem(j, NS) == s)
      def _():
        pltpu.sync_copy(zrows, acc.at[pl.ds(j * ROWB, ROWB)])
      return carry

    lax.fori_loop(0, NROWCH, zacc, 0)
    cp_idx.wait()
    plsc.subcore_barrier()

    def step(j, carry):
      jj = pl.ds(j * CH, CH)
      pltpu.async_copy(ones, acc.at[idxd.at[jj]], ssem, add=True)

      @pl.when(j >= _DEGW)
      def _():
        pltpu.make_async_copy(ones, acc.at[idxd.at[jj]], ssem).wait()
      return carry

    lax.fori_loop(0, NCHUNK, step, 0)

    def drain(j, carry):
      pltpu.make_async_copy(ones, acc.at[idxd.at[pl.ds(0, CH)]], ssem).wait()
      return carry

    lax.fori_loop(0, _DEGW, drain, 0)
    plsc.subcore_barrier()

    def cpout(j, carry):
      @pl.when(lax.rem(j, NS) == s)
      def _():
        pltpu.sync_copy(acc.at[pl.ds(j * ROWB, ROWB)], zrows)
        pltpu.sync_copy(zrows, out_hbm.at[c, pl.ds(j * ROWB, ROWB)])
      return carry

    lax.fori_loop(0, NROWCH, cpout, 0)

  return k(eidx)


def _sc_spmm(table, eidx, width, nb, ch):
  nch = EPW // ch

  @functools.partial(
      pl.kernel,
      out_type=jax.ShapeDtypeStruct((NC, N, width), jnp.float32),
      mesh=_MESH,
      compiler_params=pltpu.CompilerParams(use_tc_tiling_on_sc=False),
      scratch_types=[
          pltpu.VMEM((EPW,), jnp.int32),
          pltpu.VMEM((EPW,), jnp.int32),
      ] + [pltpu.VMEM((ch, width), jnp.float32)] * nb + [
          pltpu.VMEM_SHARED((N, width), jnp.float32),
          pltpu.SemaphoreType.DMA,
      ] + [pltpu.SemaphoreType.DMA] * (2 * nb),
  )
  def k(table_hbm, e_hbm, out_hbm, idxs, idxd, *bufs):
    rows = bufs[:nb]
    acc = bufs[nb]
    psem = bufs[nb + 1]
    g = bufs[nb + 2:nb + 2 + nb]
    st = bufs[nb + 2 + nb:]
    c = lax.axis_index("c")
    s = lax.axis_index("s")
    gid = c * NS + s

    cp_si = pltpu.async_copy(e_hbm.at[0, gid], idxs, psem)
    cp_di = pltpu.async_copy(e_hbm.at[1, gid], idxd, psem)

    _fill(rows[0], ch, width, 0.0)

    def zacc(j, carry):
      @pl.when(lax.rem(j, NS) == s)
      def _():
        pltpu.sync_copy(rows[0], acc.at[pl.ds(j * ch, ch)])
      return carry

    lax.fori_loop(0, N // ch, zacc, 0)
    cp_si.wait()
    cp_di.wait()
    for b in range(nb - 1):
      pltpu.async_copy(table_hbm.at[idxs.at[pl.ds(b * ch, ch)]], rows[b], g[b])
    plsc.subcore_barrier()

    def step(i, carry):
      for b in range(nb):
        j = nb * i + b
        js = pl.ds(j * ch, ch)
        ja = pl.ds((j + nb - 1) * ch, ch)
        pltpu.make_async_copy(table_hbm.at[idxs.at[js]], rows[b], g[b]).wait()
        bn = (b + nb - 1) % nb
        if b == 0:
          @pl.when(i > 0)
          def _():
            pltpu.make_async_copy(rows[bn], acc.at[idxd.at[js]], st[bn]).wait()
          pltpu.async_copy(table_hbm.at[idxs.at[ja]], rows[bn], g[bn])
        else:
          pltpu.make_async_copy(rows[bn], acc.at[idxd.at[js]], st[bn]).wait()

          @pl.when(i < nch // nb - 1)
          def _():
            pltpu.async_copy(table_hbm.at[idxs.at[ja]], rows[bn], g[bn])
        pltpu.async_copy(rows[b], acc.at[idxd.at[js]], st[b], add=True)
      return carry

    lax.fori_loop(0, nch // nb, step, 0)
    pltpu.make_async_copy(rows[nb - 1], acc.at[idxd.at[pl.ds(0, ch)]],
                          st[nb - 1]).wait()
    plsc.subcore_barrier()

    def cpout(j, carry):
      @pl.when(lax.rem(j, NS) == s)
      def _():
        pltpu.sync_copy(acc.at[pl.ds(j * ch, ch)], rows[0])
        pltpu.sync_copy(rows[0], out_hbm.at[c, pl.ds(j * ch, ch)])
      return carry

    lax.fori_loop(0, N // ch, cpout, 0)

  return k(table, eidx)


_NQ = OUT_CH // 16


def _sc_edge_decode(u, z, eidx, bb16):
  @functools.partial(
      pl.kernel,
      out_type=jax.ShapeDtypeStruct((E,), jnp.float32),
      mesh=_MESH,
      compiler_params=pltpu.CompilerParams(
          use_tc_tiling_on_sc=False, needs_layout_passes=False),
      scratch_types=[
          pltpu.VMEM((EPW,), jnp.int32),
          pltpu.VMEM((EPW,), jnp.int32),
          pltpu.VMEM((CH, OUT_CH), jnp.float32),
          pltpu.VMEM((CH, OUT_CH), jnp.float32),
          pltpu.VMEM((CH, OUT_CH), jnp.float32),
          pltpu.VMEM((CH, OUT_CH), jnp.float32),
          pltpu.VMEM((CH,), jnp.float32),
          pltpu.VMEM((CH,), jnp.float32),
          pltpu.VMEM((16,), jnp.float32),
      ] + [pltpu.SemaphoreType.DMA] * 7,
  )
  def k(u_hbm, z_hbm, e_hbm, bb_hbm, out_hbm,
        idxs, idxd, ubuf0, zbuf0, ubuf1, zbuf1, obuf0, obuf1, bbv,
        psem, gu0, gz0, gu1, gz1, wo0, wo1):
    c = lax.axis_index("c")
    s = lax.axis_index("s")
    gid = c * NS + s
    base = gid * EPW

    cp_si = pltpu.async_copy(e_hbm.at[0, gid], idxs, psem)
    cp_di = pltpu.async_copy(e_hbm.at[1, gid], idxd, psem)
    pltpu.sync_copy(bb_hbm, bbv)
    bias = bbv[...]
    lane = lax.iota(jnp.int32, 16)
    cp_si.wait()
    cp_di.wait()
    pltpu.async_copy(u_hbm.at[idxs.at[pl.ds(0, CH)]], ubuf0, gu0)
    pltpu.async_copy(z_hbm.at[idxd.at[pl.ds(0, CH)]], zbuf0, gz0)

    def dot_chunk(ubuf, zbuf, obuf):
      def grp(g, carry):
        res = jnp.zeros((16,), jnp.float32)
        for e in range(16):
          row = g * 16 + e
          acc = ubuf[row, pl.ds(0, 16)] * zbuf[row, pl.ds(0, 16)]
          for q in range(1, _NQ):
            acc = acc + ubuf[row, pl.ds(q * 16, 16)] * zbuf[row, pl.ds(q * 16, 16)]
          res = jnp.where(lane == e, jnp.full((16,), jnp.sum(acc)), res)
        obuf[pl.ds(g * 16, 16)] = 1.0 / (1.0 + jnp.exp(-(res + bias)))
        return carry

      lax.fori_loop(0, CH // 16, grp, 0)

    def step(i, carry):
      j0 = 2 * i
      j1 = 2 * i + 1
      off0 = pl.multiple_of(base + j0 * CH, 8)
      off1 = pl.multiple_of(base + j1 * CH, 8)
      j0s = pl.ds(j0 * CH, CH)
      j1s = pl.ds(j1 * CH, CH)
      j2s = pl.ds((j0 + 2) * CH, CH)
      pltpu.make_async_copy(u_hbm.at[idxs.at[j0s]], ubuf0, gu0).wait()
      pltpu.make_async_copy(z_hbm.at[idxd.at[j0s]], zbuf0, gz0).wait()
      pltpu.async_copy(u_hbm.at[idxs.at[j1s]], ubuf1, gu1)
      pltpu.async_copy(z_hbm.at[idxd.at[j1s]], zbuf1, gz1)

      @pl.when(i > 0)
      def _():
        pltpu.make_async_copy(obuf0, out_hbm.at[pl.ds(off0, CH)], wo0).wait()

      dot_chunk(ubuf0, zbuf0, obuf0)
      pltpu.async_copy(obuf0, out_hbm.at[pl.ds(off0, CH)], wo0)
      pltpu.make_async_copy(u_hbm.at[idxs.at[j1s]], ubuf1, gu1).wait()
      pltpu.make_async_copy(z_hbm.at[idxd.at[j1s]], zbuf1, gz1).wait()
      pltpu.async_copy(u_hbm.at[idxs.at[j2s]], ubuf0, gu0)
      pltpu.async_copy(z_hbm.at[idxd.at[j2s]], zbuf0, gz0)

      @pl.when(i > 0)
      def _():
        pltpu.make_async_copy(obuf1, out_hbm.at[pl.ds(off1, CH)], wo1).wait()

      dot_chunk(ubuf1, zbuf1, obuf1)
      pltpu.async_copy(obuf1, out_hbm.at[pl.ds(off1, CH)], wo1)
      return carry

    lax.fori_loop(0, NCHUNK // 2, step, 0)

    last = NCHUNK - 1
    offl = pl.multiple_of(base + last * CH, 8)
    lasts = pl.ds(last * CH, CH)
    pltpu.make_async_copy(u_hbm.at[idxs.at[lasts]], ubuf0, gu0).wait()
    pltpu.make_async_copy(z_hbm.at[idxd.at[lasts]], zbuf0, gz0).wait()
    pltpu.make_async_copy(obuf0, out_hbm.at[pl.ds(offl, CH)], wo0).wait()
    dot_chunk(ubuf0, zbuf0, obuf0)
    pltpu.sync_copy(obuf0, out_hbm.at[pl.ds(offl, CH)])
    pltpu.make_async_copy(obuf1, out_hbm.at[pl.ds(offl, CH)], wo1).wait()

  return k(u, z, eidx, bb16)


_NBLK = 2000
_GRID = N // _NBLK


def _dinv_from(deg_ref):
  deg = 1.0 + deg_ref[0, :, 0:1] + deg_ref[1, :, 0:1]
  return lax.rsqrt(deg)


_DEGSPEC = pl.BlockSpec((NC, _NBLK, 16), lambda i: (0, i, 0))


def _tc_a(x, W1, deg2):
  def body(x_ref, w_ref, d_ref, q_ref):
    dinv = _dinv_from(d_ref)
    p = jnp.dot(x_ref[:, :], w_ref[:, :], preferred_element_type=jnp.float32)
    q_ref[:, :] = dinv * p

  return pl.pallas_call(
      body,
      grid=(_GRID,),
      in_specs=[
          pl.BlockSpec((_NBLK, IN_CH), lambda i: (i, 0)),
          pl.BlockSpec((IN_CH, HID), lambda i: (0, 0)),
          _DEGSPEC,
      ],
      out_specs=pl.BlockSpec((_NBLK, HID), lambda i: (i, 0)),
      out_shape=jax.ShapeDtypeStruct((N, HID), jnp.float32),
  )(x, W1, deg2)


def _tc_c(s1, q1, deg2, b1, W2):
  def body(s_ref, q_ref, d_ref, bias_ref, w_ref, out_ref):
    dinv = _dinv_from(d_ref)
    h = dinv * (s_ref[0] + s_ref[1] + q_ref[:, :]) + bias_ref[:, :]
    h = jnp.maximum(h, 0.0)
    p2 = jnp.dot(h, w_ref[:, :], preferred_element_type=jnp.float32)
    out_ref[:, :] = dinv * p2

  return pl.pallas_call(
      body,
      grid=(_GRID,),
      in_specs=[
          pl.BlockSpec((NC, _NBLK, HID), lambda i: (0, i, 0)),
          pl.BlockSpec((_NBLK, HID), lambda i: (i, 0)),
          _DEGSPEC,
          pl.BlockSpec((1, HID), lambda i: (0, 0)),
          pl.BlockSpec((HID, OUT_CH), lambda i: (0, 0)),
      ],
      out_specs=pl.BlockSpec((_NBLK, OUT_CH), lambda i: (i, 0)),
      out_shape=jax.ShapeDtypeStruct((N, OUT_CH), jnp.float32),
  )(s1, q1, deg2, b1, W2)


def _tc_e(s2, q2, deg2, b2, Wb0):
  def body(s_ref, q_ref, d_ref, bias_ref, w_ref, z_ref, u_ref):
    dinv = _dinv_from(d_ref)
    z = dinv * (s_ref[0] + s_ref[1] + q_ref[:, :]) + bias_ref[:, :]
    z_ref[:, :] = z
    u_ref[:, :] = jnp.dot(z, w_ref[:, :], preferred_element_type=jnp.float32)

  return pl.pallas_call(
      body,
      grid=(_GRID,),
      in_specs=[
          pl.BlockSpec((NC, _NBLK, OUT_CH), lambda i: (0, i, 0)),
          pl.BlockSpec((_NBLK, OUT_CH), lambda i: (i, 0)),
          _DEGSPEC,
          pl.BlockSpec((1, OUT_CH), lambda i: (0, 0)),
          pl.BlockSpec((OUT_CH, OUT_CH), lambda i: (0, 0)),
      ],
      out_specs=[
          pl.BlockSpec((_NBLK, OUT_CH), lambda i: (i, 0)),
          pl.BlockSpec((_NBLK, OUT_CH), lambda i: (i, 0)),
      ],
      out_shape=[
          jax.ShapeDtypeStruct((N, OUT_CH), jnp.float32),
          jax.ShapeDtypeStruct((N, OUT_CH), jnp.float32),
      ],
  )(s2, q2, deg2, b2, Wb0)


def kernel(x, edge_index, W1, b1, W2, b2, Wb, bb):
  eidx = edge_index.reshape(2, NW, EPW)

  deg2 = _sc_deg(eidx)
  q1 = _tc_a(x, W1, deg2)
  s1 = _sc_spmm(q1, eidx, HID, 5, 40)
  q2 = _tc_c(s1, q1, deg2, b1.reshape(1, HID), W2)
  s2 = _sc_spmm(q2, eidx, OUT_CH, 5, 80)
  z, u = _tc_e(s2, q2, deg2, b2.reshape(1, OUT_CH), Wb[0])
  bb16 = jnp.broadcast_to(bb.reshape(1), (16,))
  return _sc_edge_decode(u, z, eidx, bb16).reshape(E, 1)
